# baseline (speedup 1.0000x reference)
import jax
import jax.numpy as jnp

import kernel_probe_lib as L


def kernel(kjt_ids, tables, W1, b1, W2, b2, W3, b3):
    nf, vocab, dim = tables.shape
    ids_flat = kjt_ids.reshape(-1).astype(jnp.int32)
    tab_cm = tables.transpose(0, 2, 1).reshape(nf * dim, vocab)
    tab_im = L._tc_transpose(tab_cm, nf, vocab, dim).reshape(nf * vocab, dim)
    gath = L._sc_gather(tab_im, ids_flat, vocab)
    return gath[:16384, 0]
